# baseline (speedup 1.0000x reference)
import functools

import jax
import jax.numpy as jnp
from jax import lax
from jax.experimental import pallas as pl
from jax.experimental.pallas import tpu as pltpu
from jax.experimental.pallas import tpu_sc as plsc

BATCH = 16384
EMBED = 32
N_USERS = 1000000
_INFO = plsc.get_sparse_core_info()
NC, NS, L = _INFO.num_cores, _INFO.num_subcores, _INFO.num_lanes
NW = NC * NS
B_PER_W = BATCH // NW
N_CHUNK = 1
CHUNK = B_PER_W // N_CHUNK

PROJ_BLK = 65536


def _tc_proj_body(w_ref, t_ref, o_ref):
    w = w_ref[...]
    prod = jnp.dot(w, t_ref[...],
                   preferred_element_type=jnp.float32)
    o_ref[...] = prod.reshape(o_ref.shape[0])


def _tc_project(table_t, w_row):
    n = table_t.shape[1]
    grid = (n + PROJ_BLK - 1) // PROJ_BLK
    return pl.pallas_call(
        _tc_proj_body,
        grid=(grid,),
        in_specs=[
            pl.BlockSpec((1, EMBED), lambda i: (0, 0)),
            pl.BlockSpec((EMBED, PROJ_BLK), lambda i: (0, i)),
        ],
        out_specs=pl.BlockSpec((PROJ_BLK,), lambda i: (i,)),
        out_shape=jax.ShapeDtypeStruct((n,), jnp.float32),
    )(w_row, table_t)


def _sc_body(users_hbm, autos_hbm, pu_hbm, pa_hbm, b_hbm,
             out_hbm, idx_u, idx_a, gu, ga, out_v, b_v,
             sem, sem_u, sem_a):
    wid = lax.axis_index("s") * NC + lax.axis_index("c")

    cu = pltpu.async_copy(users_hbm.at[wid], idx_u, sem_u)
    ca = pltpu.async_copy(autos_hbm.at[wid], idx_a, sem_a)
    pltpu.sync_copy(b_hbm, b_v)

    copies = []
    cu.wait()
    for j in range(N_CHUNK):
        copies.append(pltpu.async_copy(
            pu_hbm.at[idx_u.at[j]], gu.at[pl.ds(j * CHUNK, CHUNK)], sem))
    ca.wait()
    for j in range(N_CHUNK):
        copies.append(pltpu.async_copy(
            pa_hbm.at[idx_a.at[j]], ga.at[pl.ds(j * CHUNK, CHUNK)], sem))
    for c in copies:
        c.wait()

    bvec = b_v[...]

    def body(k, carry):
        sl = pl.ds(k * L, L)
        out_v[sl] = gu[sl] + ga[sl] + bvec
        return carry

    lax.fori_loop(0, B_PER_W // L, body, 0)

    pltpu.sync_copy(out_v, out_hbm.at[pl.ds(wid * B_PER_W, B_PER_W)])


def _sc_gather(users_r, autos_r, pu, pa, b16):
    mesh = plsc.VectorSubcoreMesh(core_axis_name="c", subcore_axis_name="s")
    f = functools.partial(
        pl.kernel, mesh=mesh,
        compiler_params=pltpu.CompilerParams(needs_layout_passes=False,
                                             use_tc_tiling_on_sc=False),
        out_type=jax.ShapeDtypeStruct((BATCH,), jnp.float32),
        scratch_types=[
            pltpu.VMEM((N_CHUNK, CHUNK), jnp.int32),
            pltpu.VMEM((N_CHUNK, CHUNK), jnp.int32),
            pltpu.VMEM((B_PER_W,), jnp.float32),
            pltpu.VMEM((B_PER_W,), jnp.float32),
            pltpu.VMEM((B_PER_W,), jnp.float32),
            pltpu.VMEM((L,), jnp.float32),
            pltpu.SemaphoreType.DMA,
            pltpu.SemaphoreType.DMA,
            pltpu.SemaphoreType.DMA,
        ],
    )(_sc_body)
    return f(users_r, autos_r, pu, pa, b16)


@jax.jit
def _run(users_r, autos_r, user_table_t, auto_table_t, W, b):
    wf = W.astype(jnp.float32)
    wu = wf[:EMBED].reshape(1, EMBED)
    wa = wf[EMBED:].reshape(1, EMBED)

    pu = _tc_project(user_table_t, wu)
    pa = _tc_project(auto_table_t, wa)

    b16 = jnp.broadcast_to(b.astype(jnp.float32), (L,))
    return _sc_gather(users_r, autos_r, pu, pa, b16)


def kernel(users, autos, user_table, auto_table, W, b):
    users_r = users.astype(jnp.int32).reshape(NW, N_CHUNK, CHUNK)
    autos_r = autos.astype(jnp.int32).reshape(NW, N_CHUNK, CHUNK)
    out = _run(users_r, autos_r, user_table.T, auto_table.T, W, b)
    return out.reshape(BATCH, 1)


# speedup vs baseline: 1.0094x; 1.0094x over previous
import functools

import jax
import jax.numpy as jnp
from jax import lax
from jax.experimental import pallas as pl
from jax.experimental.pallas import tpu as pltpu
from jax.experimental.pallas import tpu_sc as plsc

BATCH = 16384
EMBED = 32
N_USERS = 1000000
_INFO = plsc.get_sparse_core_info()
NC, NS, L = _INFO.num_cores, _INFO.num_subcores, _INFO.num_lanes
NW = NC * NS
B_PER_W = BATCH // NW
N_CHUNK = 2
CHUNK = B_PER_W // N_CHUNK

PROJ_BLK = 65536


def _tc_proj_body(w_ref, b_ref, t_ref, o_ref):
    w = w_ref[...]
    prod = jnp.dot(w, t_ref[...],
                   preferred_element_type=jnp.float32)
    o_ref[...] = prod.reshape(o_ref.shape[0]) + b_ref[0, 0]


def _tc_project(table_t, w_row, bias11):
    n = table_t.shape[1]
    grid = (n + PROJ_BLK - 1) // PROJ_BLK
    return pl.pallas_call(
        _tc_proj_body,
        grid=(grid,),
        in_specs=[
            pl.BlockSpec((1, EMBED), lambda i: (0, 0)),
            pl.BlockSpec((1, 1), lambda i: (0, 0)),
            pl.BlockSpec((EMBED, PROJ_BLK), lambda i: (0, i)),
        ],
        out_specs=pl.BlockSpec((PROJ_BLK,), lambda i: (i,)),
        out_shape=jax.ShapeDtypeStruct((n,), jnp.float32),
    )(w_row, bias11, table_t)


def _sc_body(users_hbm, autos_hbm, pu_hbm, pa_hbm,
             out_hbm, idx_u, idx_a, gu, ga, out_v,
             sem, sem_u, sem_a):
    wid = lax.axis_index("s") * NC + lax.axis_index("c")

    cu = pltpu.async_copy(users_hbm.at[wid], idx_u, sem_u)
    ca = pltpu.async_copy(autos_hbm.at[wid], idx_a, sem_a)

    copies = []
    cu.wait()
    for j in range(N_CHUNK):
        copies.append(pltpu.async_copy(
            pu_hbm.at[idx_u.at[j]], gu.at[pl.ds(j * CHUNK, CHUNK)], sem))
    ca.wait()
    for j in range(N_CHUNK):
        copies.append(pltpu.async_copy(
            pa_hbm.at[idx_a.at[j]], ga.at[pl.ds(j * CHUNK, CHUNK)], sem))
    for c in copies:
        c.wait()

    def body(k, carry):
        sl = pl.ds(k * L, L)
        out_v[sl] = gu[sl] + ga[sl]
        return carry

    lax.fori_loop(0, B_PER_W // L, body, 0)

    pltpu.sync_copy(out_v, out_hbm.at[pl.ds(wid * B_PER_W, B_PER_W)])


def _sc_gather(users_r, autos_r, pu, pa):
    mesh = plsc.VectorSubcoreMesh(core_axis_name="c", subcore_axis_name="s")
    f = functools.partial(
        pl.kernel, mesh=mesh,
        compiler_params=pltpu.CompilerParams(needs_layout_passes=False,
                                             use_tc_tiling_on_sc=False),
        out_type=jax.ShapeDtypeStruct((BATCH,), jnp.float32),
        scratch_types=[
            pltpu.VMEM((N_CHUNK, CHUNK), jnp.int32),
            pltpu.VMEM((N_CHUNK, CHUNK), jnp.int32),
            pltpu.VMEM((B_PER_W,), jnp.float32),
            pltpu.VMEM((B_PER_W,), jnp.float32),
            pltpu.VMEM((B_PER_W,), jnp.float32),
            pltpu.SemaphoreType.DMA,
            pltpu.SemaphoreType.DMA,
            pltpu.SemaphoreType.DMA,
        ],
    )(_sc_body)
    return f(users_r, autos_r, pu, pa)


@jax.jit
def _run(users_r, autos_r, user_table_t, auto_table_t, W, b):
    wf = W.astype(jnp.float32)
    wu = wf[:EMBED].reshape(1, EMBED)
    wa = wf[EMBED:].reshape(1, EMBED)
    zero11 = jnp.zeros((1, 1), jnp.float32)
    b11 = b.astype(jnp.float32).reshape(1, 1)

    pu = _tc_project(user_table_t, wu, zero11)
    pa = _tc_project(auto_table_t, wa, b11)

    return _sc_gather(users_r, autos_r, pu, pa)


def kernel(users, autos, user_table, auto_table, W, b):
    users_r = users.astype(jnp.int32).reshape(NW, N_CHUNK, CHUNK)
    autos_r = autos.astype(jnp.int32).reshape(NW, N_CHUNK, CHUNK)
    out = _run(users_r, autos_r, user_table.T, auto_table.T, W, b)
    return out.reshape(BATCH, 1)
